# Initial kernel scaffold; baseline (speedup 1.0000x reference)
import jax
import jax.numpy as jnp
from jax.experimental import pallas as pl


def kernel(cloud_data, params):
    raise NotImplementedError("write your pallas kernel here")


# speedup vs baseline: 1.1882x; 1.1882x over previous
import functools

import jax
import jax.numpy as jnp
from jax.experimental import pallas as pl

_BN_EPS = 1e-5
_B, _N = 8, 4096
_K = 32
_NEG = -1e30


def _fps_body(m, pos_ref, out_ref):
    px = pos_ref[:, 0, :]
    py = pos_ref[:, 1, :]
    pz = pos_ref[:, 2, :]
    b, n = px.shape
    iota = jax.lax.broadcasted_iota(jnp.int32, (b, n), 1)
    iota_m = jax.lax.broadcasted_iota(jnp.int32, (b, m), 1)

    dist0 = ((px - px[:, 0:1]) ** 2 + (py - py[:, 0:1]) ** 2
             + (pz - pz[:, 0:1]) ** 2)
    idxs0 = jnp.zeros((b, m), jnp.int32)

    def body(i, carry):
        dist, idxs = carry
        mx = jnp.max(dist, axis=1, keepdims=True)
        nxt = jnp.min(jnp.where(dist == mx, iota, n), axis=1)
        onehot = iota == nxt[:, None]
        nx = jnp.sum(jnp.where(onehot, px, 0.0), axis=1)
        ny = jnp.sum(jnp.where(onehot, py, 0.0), axis=1)
        nz = jnp.sum(jnp.where(onehot, pz, 0.0), axis=1)
        d = ((px - nx[:, None]) ** 2 + (py - ny[:, None]) ** 2
             + (pz - nz[:, None]) ** 2)
        dist = jnp.minimum(dist, d)
        idxs = jnp.where(iota_m == i, nxt[:, None], idxs)
        return dist, idxs

    _, idxs = jax.lax.fori_loop(1, m, body, (dist0, idxs0))
    out_ref[...] = idxs


def _fps(pos3, m):
    b = pos3.shape[0]
    return pl.pallas_call(
        functools.partial(_fps_body, m),
        out_shape=jax.ShapeDtypeStruct((b, m), jnp.int32),
    )(pos3)


def _layers(x, refs):
    for (w_r, b_r, s_r, be_r) in refs:
        x = jnp.dot(x, w_r[...], preferred_element_type=jnp.float32)
        x = jax.nn.relu(x + b_r[...])
        x = x * s_r[...] + be_r[...]
    return x


def _prep_params(layers):
    out = []
    for l in layers:
        s = l["gamma"] / jnp.sqrt(1.0 + _BN_EPS)
        out.extend([l["W"], l["b"][None, :], s[None, :], l["beta"][None, :]])
    return out


def _full(a):
    return pl.BlockSpec(a.shape, lambda *_: (0,) * a.ndim)


def _sa_body(k, *refs):
    x_ref, madd_ref = refs[0], refs[1]
    out_ref = refs[-1]
    prm = [tuple(refs[2 + 4 * j: 6 + 4 * j]) for j in range(2)]
    x = _layers(x_ref[...], prm)
    mt = madd_ref.shape[0]
    msg = x.reshape(mt, k, x.shape[-1]) + madd_ref[...][:, :, None]
    out_ref[...] = jnp.max(msg, axis=1)


def _sa_mlp_max(xcat, madd, layers, mt):
    bm, k = madd.shape
    cin = xcat.shape[-1]
    cout = layers[-1]["W"].shape[-1]
    prm = _prep_params(layers)
    grid = (bm // mt,)
    in_specs = [
        pl.BlockSpec((mt * k, cin), lambda i: (i, 0)),
        pl.BlockSpec((mt, k), lambda i: (i, 0)),
    ] + [_full(a) for a in prm]
    return pl.pallas_call(
        functools.partial(_sa_body, k),
        grid=grid,
        in_specs=in_specs,
        out_specs=pl.BlockSpec((mt, cout), lambda i: (i, 0)),
        out_shape=jax.ShapeDtypeStruct((bm, cout), jnp.float32),
    )(xcat, madd, *prm)


def _sa3_body(*refs):
    x_ref, out_ref = refs[0], refs[-1]
    prm = [tuple(refs[1 + 4 * j: 5 + 4 * j]) for j in range(2)]
    h = _layers(x_ref[0], prm)
    out_ref[0] = jnp.max(h, axis=0, keepdims=True)


def _sa3_globalmax(x, layers):
    b, n, c = x.shape
    cout = layers[-1]["W"].shape[-1]
    prm = _prep_params(layers)
    in_specs = [pl.BlockSpec((1, n, c), lambda i: (i, 0, 0))]
    in_specs += [_full(a) for a in prm]
    g = pl.pallas_call(
        _sa3_body,
        grid=(b,),
        in_specs=in_specs,
        out_specs=pl.BlockSpec((1, 1, cout), lambda i: (i, 0, 0)),
        out_shape=jax.ShapeDtypeStruct((b, 1, cout), jnp.float32),
    )(x, *prm)
    return g.reshape(b, cout)


def _fp3_body(*refs):
    g_ref, sk_ref, out_ref = refs[0], refs[1], refs[-1]
    prm = [tuple(refs[2 + 4 * j: 6 + 4 * j]) for j in range(2)]
    x = jnp.concatenate([g_ref[...], sk_ref[...]], axis=1)
    out_ref[...] = _layers(x, prm)


def _fp3(gb, skip, layers):
    r = gb.shape[0]
    cout = layers[-1]["W"].shape[-1]
    prm = _prep_params(layers)
    return pl.pallas_call(
        _fp3_body,
        in_specs=[_full(gb), _full(skip)] + [_full(a) for a in prm],
        out_specs=pl.BlockSpec((r, cout), lambda *_: (0, 0)),
        out_shape=jax.ShapeDtypeStruct((r, cout), jnp.float32),
    )(gb, skip, *prm)


def _interp3(sel_ref, w_ref, c):
    w = w_ref[...]
    sel = sel_ref[...]
    num = (w[:, 0:1] * sel[:, :c] + w[:, 1:2] * sel[:, c:2 * c]
           + w[:, 2:3] * sel[:, 2 * c:])
    den = w[:, 0:1] + w[:, 1:2] + w[:, 2:3]
    return num / den


def _fp2_body(c, *refs):
    sel_ref, w_ref, sk_ref, out_ref = refs[0], refs[1], refs[2], refs[-1]
    prm = [tuple(refs[3 + 4 * j: 7 + 4 * j]) for j in range(2)]
    y = _interp3(sel_ref, w_ref, c)
    x = jnp.concatenate([y, sk_ref[...]], axis=1)
    out_ref[...] = _layers(x, prm)


def _fp2(sel, w, skip, layers, t):
    r, c3 = sel.shape
    c = c3 // 3
    cout = layers[-1]["W"].shape[-1]
    prm = _prep_params(layers)
    in_specs = [
        pl.BlockSpec((t, c3), lambda i: (i, 0)),
        pl.BlockSpec((t, 3), lambda i: (i, 0)),
        pl.BlockSpec((t, skip.shape[-1]), lambda i: (i, 0)),
    ] + [_full(a) for a in prm]
    return pl.pallas_call(
        functools.partial(_fp2_body, c),
        grid=(r // t,),
        in_specs=in_specs,
        out_specs=pl.BlockSpec((t, cout), lambda i: (i, 0)),
        out_shape=jax.ShapeDtypeStruct((r, cout), jnp.float32),
    )(sel, w, skip, *prm)


def _fp1_body(c, *refs):
    sel_ref, w_ref, sk_ref = refs[0], refs[1], refs[2]
    prm = [tuple(refs[3 + 4 * j: 7 + 4 * j]) for j in range(2)]
    w1_ref, b1_ref, w2_ref, b2_ref = refs[11:15]
    sc_ref, pr_ref = refs[-2], refs[-1]
    y = _interp3(sel_ref, w_ref, c)
    x = jnp.concatenate([y, sk_ref[...]], axis=1)
    x = _layers(x, prm)
    h = jax.nn.relu(jnp.dot(x, w1_ref[...], preferred_element_type=jnp.float32)
                    + b1_ref[...])
    s = jnp.dot(h, w2_ref[...], preferred_element_type=jnp.float32) + b2_ref[...]
    sc_ref[...] = s
    e = jnp.exp(s - jnp.max(s, axis=1, keepdims=True))
    pr_ref[...] = e / jnp.sum(e, axis=1, keepdims=True)


def _fp1_head(sel, w, skip, layers, lin1, lin2, t):
    r, c3 = sel.shape
    c = c3 // 3
    prm = _prep_params(layers)
    head = [lin1["W"], lin1["b"][None, :], lin2["W"], lin2["b"][None, :]]
    ncls = lin2["W"].shape[-1]
    in_specs = [
        pl.BlockSpec((t, c3), lambda i: (i, 0)),
        pl.BlockSpec((t, 3), lambda i: (i, 0)),
        pl.BlockSpec((t, skip.shape[-1]), lambda i: (i, 0)),
    ] + [_full(a) for a in prm + head]
    return pl.pallas_call(
        functools.partial(_fp1_body, c),
        grid=(r // t,),
        in_specs=in_specs,
        out_specs=[pl.BlockSpec((t, ncls), lambda i: (i, 0))] * 2,
        out_shape=[jax.ShapeDtypeStruct((r, ncls), jnp.float32)] * 2,
    )(sel, w, skip, *prm, *head)


def _ball_query(pos_s, pos, r, k):
    d2 = jnp.sum((pos_s[:, :, None, :] - pos[:, None, :, :]) ** 2, -1)
    score = jnp.where(d2 < r * r, -d2, -jnp.inf)
    top_score, nbr = jax.lax.top_k(score, k)
    valid = top_score > -jnp.inf
    nbr = jnp.where(valid, nbr, 0)
    return nbr, valid


def _gather_rows(x, idx):
    b = x.shape[0]
    flat = idx.reshape(b, -1)
    return jnp.take_along_axis(x, flat[:, :, None], axis=1)


def _sa_stage(x, pos, pos3, ratio, r, layers, mt):
    b, n, _ = pos.shape
    m = int(ratio * n)
    idx = _fps(pos3, m)
    pos_s = _gather_rows(pos, idx).reshape(b, m, 3)
    nbr, valid = _ball_query(pos_s, pos, r, _K)
    cin = x.shape[-1]
    x_j = _gather_rows(x, nbr).reshape(b, m, _K, cin)
    pos_j = _gather_rows(pos, nbr).reshape(b, m, _K, 3)
    xcat = jnp.concatenate([x_j, pos_j - pos_s[:, :, None, :]], -1)
    madd = jnp.where(valid, 0.0, _NEG).astype(jnp.float32)
    out = _sa_mlp_max(xcat.reshape(b * m * _K, cin + 3),
                      madd.reshape(b * m, _K), layers, mt)
    return out.reshape(b, m, -1), pos_s


def _knn3(pos_dst, pos_src):
    d2 = jnp.sum((pos_dst[:, :, None, :] - pos_src[:, None, :, :]) ** 2, -1)
    neg_d, idx = jax.lax.top_k(-d2, 3)
    w = 1.0 / jnp.clip(-neg_d, 1e-16)
    return idx, w


def kernel(cloud_data, params):
    b = cloud_data.shape[0]
    n = cloud_data.shape[2]
    pos3 = cloud_data[:, :3, :]
    xyz = jnp.transpose(pos3, (0, 2, 1))
    feat = jnp.transpose(cloud_data[:, 2:9, :], (0, 2, 1))

    x1, pos1 = _sa_stage(feat, xyz, pos3, 0.25, 0.2, params["sa1"], 512)
    pos1_t = jnp.transpose(pos1, (0, 2, 1))
    x2, pos2 = _sa_stage(x1, pos1, pos1_t, 0.25, 0.4, params["sa2"], 512)
    m1, m2 = pos1.shape[1], pos2.shape[1]

    g = _sa3_globalmax(jnp.concatenate([x2, pos2], -1), params["sa3"])

    gb = jnp.broadcast_to(g[:, None, :], (b, m2, g.shape[-1]))
    f3 = _fp3(gb.reshape(b * m2, -1), x2.reshape(b * m2, -1), params["fp3"])

    idx2, w2 = _knn3(pos1, pos2)
    sel2 = _gather_rows(f3.reshape(b, m2, -1), idx2).reshape(b * m1, -1)
    f2 = _fp2(sel2, w2.reshape(b * m1, 3), x1.reshape(b * m1, -1),
              params["fp2"], 1024)

    idx1, w1 = _knn3(xyz, pos1)
    sel1 = _gather_rows(f2.reshape(b, m1, -1), idx1).reshape(b * n, -1)
    scores, proba = _fp1_head(sel1, w1.reshape(b * n, 3),
                              feat.reshape(b * n, -1), params["fp1"],
                              params["lin1"], params["lin2"], 2048)

    ncls = scores.shape[-1]
    scores = jnp.transpose(scores.reshape(b, n, ncls), (0, 2, 1))
    proba = jnp.transpose(proba.reshape(b, n, ncls), (0, 2, 1))
    return scores, proba
